# Initial kernel scaffold; baseline (speedup 1.0000x reference)
import jax
import jax.numpy as jnp
from jax.experimental import pallas as pl


def kernel(positions, adj, Ws, bs):
    raise NotImplementedError("write your pallas kernel here")


# speedup vs baseline: 1.5650x; 1.5650x over previous
import jax
import jax.numpy as jnp
from jax.experimental import pallas as pl
from jax.experimental.pallas import tpu as pltpu


def _encoder_kernel(*refs):
    pos_ref, adj_ref = refs[0], refs[1]
    out_ref = refs[-1]
    n_layers = (len(refs) - 3) // 2
    w_refs = refs[2:2 + n_layers]
    b_refs = refs[2 + n_layers:2 + 2 * n_layers]

    adj = adj_ref[...]
    x = pos_ref[...]
    for i in range(n_layers):
        w = w_refs[i][...]
        b = b_refs[i][...]
        support = jnp.dot(x, w, preferred_element_type=jnp.float32)
        agg = jnp.dot(adj, support, preferred_element_type=jnp.float32) + b
        x = jnp.where(agg > 0, agg, jnp.exp(jnp.minimum(agg, 0.0)) - 1.0)
    out_ref[...] = jnp.max(x, axis=0, keepdims=True)


def kernel(positions, adj, Ws, bs):
    bs2 = [b.reshape(1, -1) for b in bs]
    out = pl.pallas_call(
        _encoder_kernel,
        out_shape=jax.ShapeDtypeStruct((1, Ws[-1].shape[1]), jnp.float32),
        compiler_params=pltpu.CompilerParams(
            vmem_limit_bytes=128 * 1024 * 1024,
        ),
    )(positions, adj, *Ws, *bs2)
    return out.reshape(-1)


# speedup vs baseline: 1.5948x; 1.0190x over previous
import jax
import jax.numpy as jnp
from jax.experimental import pallas as pl
from jax.experimental.pallas import tpu as pltpu


def _pad128(d):
    return ((d + 127) // 128) * 128


def _encoder_kernel(*refs):
    pos_ref, adj_ref = refs[0], refs[1]
    out_ref = refs[-1]
    n_layers = (len(refs) - 3) // 2
    w_refs = refs[2:2 + n_layers]
    b_refs = refs[2 + n_layers:2 + 2 * n_layers]

    adj = adj_ref[...].astype(jnp.bfloat16)
    x = pos_ref[...]
    for i in range(n_layers):
        w = w_refs[i][...].astype(jnp.bfloat16)
        b = b_refs[i][...]
        din, dout = w_refs[i].shape
        xb = x.astype(jnp.bfloat16)
        if _pad128(din) < _pad128(dout):
            h = jnp.dot(adj, xb, preferred_element_type=jnp.float32)
            agg = jnp.dot(h.astype(jnp.bfloat16), w,
                          preferred_element_type=jnp.float32) + b
        else:
            s = jnp.dot(xb, w, preferred_element_type=jnp.float32)
            agg = jnp.dot(adj, s.astype(jnp.bfloat16),
                          preferred_element_type=jnp.float32) + b
        x = jnp.where(agg > 0, agg, jnp.exp(jnp.minimum(agg, 0.0)) - 1.0)
    out_ref[...] = jnp.max(x, axis=0, keepdims=True)


def kernel(positions, adj, Ws, bs):
    bs2 = [b.reshape(1, -1) for b in bs]
    out = pl.pallas_call(
        _encoder_kernel,
        out_shape=jax.ShapeDtypeStruct((1, Ws[-1].shape[1]), jnp.float32),
        compiler_params=pltpu.CompilerParams(
            vmem_limit_bytes=128 * 1024 * 1024,
        ),
    )(positions, adj, *Ws, *bs2)
    return out.reshape(-1)


# speedup vs baseline: 1.5996x; 1.0030x over previous
import jax
import jax.numpy as jnp
from jax.experimental import pallas as pl
from jax.experimental.pallas import tpu as pltpu

_N = 2562
_CHUNK = 432


def _pad128(d):
    return ((d + 127) // 128) * 128


def _chunks():
    out = []
    off = 0
    while off < _N:
        out.append((off, min(_CHUNK, _N - off)))
        off += _CHUNK
    return out


def _elu(v):
    return jnp.where(v > 0, v, jnp.exp(jnp.minimum(v, 0.0)) - 1.0)


def _encoder_kernel(*refs):
    pos_ref, adj_ref = refs[0], refs[1]
    n_layers = (len(refs) - 6) // 2
    w_refs = refs[2:2 + n_layers]
    b_refs = refs[2 + n_layers:2 + 2 * n_layers]
    out_ref = refs[-4]
    adj_bf = refs[-3]
    bufs = (refs[-2], refs[-1])

    dims = [w.shape for w in w_refs]
    reassoc = [_pad128(din) < _pad128(dout) for din, dout in dims]

    adj_bf[...] = adj_ref[...].astype(jnp.bfloat16)

    s0 = jnp.dot(pos_ref[...].astype(jnp.bfloat16),
                 w_refs[0][...].astype(jnp.bfloat16),
                 preferred_element_type=jnp.float32)
    bufs[0][:, 0:dims[0][1]] = s0.astype(jnp.bfloat16)

    acc = None
    for i in range(n_layers):
        src, dst = bufs[i % 2], bufs[(i + 1) % 2]
        din, dout = dims[i]
        w_bf = w_refs[i][...].astype(jnp.bfloat16)
        b = b_refs[i][...]
        in_w = din if reassoc[i] else dout
        carry = src[:, 0:in_w]
        if i + 1 < n_layers:
            w_next = w_refs[i + 1][...].astype(jnp.bfloat16)
        for off, sz in _chunks():
            a_r = adj_bf[pl.ds(off, sz), :]
            if reassoc[i]:
                h = jnp.dot(a_r, carry, preferred_element_type=jnp.float32)
                agg = jnp.dot(h.astype(jnp.bfloat16), w_bf,
                              preferred_element_type=jnp.float32)
            else:
                agg = jnp.dot(a_r, carry, preferred_element_type=jnp.float32)
            xr = _elu(agg + b)
            if i + 1 < n_layers:
                if reassoc[i + 1]:
                    dst[pl.ds(off, sz), 0:dout] = xr.astype(jnp.bfloat16)
                else:
                    s_next = jnp.dot(xr.astype(jnp.bfloat16), w_next,
                                     preferred_element_type=jnp.float32)
                    dst[pl.ds(off, sz), 0:dims[i + 1][1]] = (
                        s_next.astype(jnp.bfloat16))
            else:
                m = jnp.max(xr, axis=0, keepdims=True)
                acc = m if acc is None else jnp.maximum(acc, m)
    out_ref[...] = acc


def kernel(positions, adj, Ws, bs):
    bs2 = [b.reshape(1, -1) for b in bs]
    max_w = max(max(d) for d in (w.shape for w in Ws))
    out = pl.pallas_call(
        _encoder_kernel,
        out_shape=jax.ShapeDtypeStruct((1, Ws[-1].shape[1]), jnp.float32),
        scratch_shapes=[
            pltpu.VMEM((_N, _N), jnp.bfloat16),
            pltpu.VMEM((_N, _pad128(max_w)), jnp.bfloat16),
            pltpu.VMEM((_N, _pad128(max_w)), jnp.bfloat16),
        ],
        compiler_params=pltpu.CompilerParams(
            vmem_limit_bytes=128 * 1024 * 1024,
        ),
    )(positions, adj, *Ws, *bs2)
    return out.reshape(-1)


# speedup vs baseline: 1.6362x; 1.0229x over previous
import jax
import jax.numpy as jnp
from jax.experimental import pallas as pl
from jax.experimental.pallas import tpu as pltpu

_N = 2562
_CHUNK = 432


def _pad128(d):
    return ((d + 127) // 128) * 128


def _chunks():
    out = []
    off = 0
    while off < _N:
        out.append((off, min(_CHUNK, _N - off)))
        off += _CHUNK
    return out


def _elu(v):
    return jnp.where(v > 0, v, jnp.exp(jnp.minimum(v, 0.0)) - 1.0)


def _encoder_kernel(*refs):
    pos_ref, adj_hbm = refs[0], refs[1]
    n_layers = (len(refs) - 10) // 2
    w_refs = refs[2:2 + n_layers]
    b_refs = refs[2 + n_layers:2 + 2 * n_layers]
    out_ref = refs[2 + 2 * n_layers]
    adj_bf = refs[-7]
    bufs = (refs[-6], refs[-5])
    stages = (refs[-4], refs[-3], refs[-2])
    sems = refs[-1]

    dims = [w.shape for w in w_refs]
    reassoc = [_pad128(din) < _pad128(dout) for din, dout in dims]
    chunks = _chunks()

    w_bf = [w_refs[i][...].astype(jnp.bfloat16) for i in range(n_layers)]

    last = len(chunks) - 1

    def stage_of(r):
        return stages[2] if r == last else stages[r % 2]

    def start_copy(r):
        off, sz = chunks[r]
        cp = pltpu.make_async_copy(
            adj_hbm.at[pl.ds(off, sz), :],
            stage_of(r),
            sems.at[2 if r == last else r % 2])
        cp.start()
        return cp

    s0 = jnp.dot(pos_ref[...].astype(jnp.bfloat16), w_bf[0],
                 preferred_element_type=jnp.float32).astype(jnp.bfloat16)
    b0 = b_refs[0][...]

    cps = [None] * len(chunks)
    cps[0] = start_copy(0)
    for r, (off, sz) in enumerate(chunks):
        if r + 1 < len(chunks):
            cps[r + 1] = start_copy(r + 1)
        cps[r].wait()
        a_r = stage_of(r)[...].astype(jnp.bfloat16)
        adj_bf[pl.ds(off, sz), :] = a_r
        agg = jnp.dot(a_r, s0, preferred_element_type=jnp.float32)
        xr = _elu(agg + b0)
        if reassoc[1]:
            bufs[1][pl.ds(off, sz), 0:dims[0][1]] = xr.astype(jnp.bfloat16)
        else:
            s_next = jnp.dot(xr.astype(jnp.bfloat16), w_bf[1],
                             preferred_element_type=jnp.float32)
            bufs[1][pl.ds(off, sz), 0:dims[1][1]] = s_next.astype(jnp.bfloat16)

    acc = None
    for i in range(1, n_layers):
        src, dst = bufs[i % 2], bufs[(i + 1) % 2]
        din, dout = dims[i]
        b = b_refs[i][...]
        in_w = din if reassoc[i] else dout
        carry = src[:, 0:in_w]
        for off, sz in chunks:
            a_r = adj_bf[pl.ds(off, sz), :]
            if reassoc[i]:
                h = jnp.dot(a_r, carry, preferred_element_type=jnp.float32)
                agg = jnp.dot(h.astype(jnp.bfloat16), w_bf[i],
                              preferred_element_type=jnp.float32)
            else:
                agg = jnp.dot(a_r, carry, preferred_element_type=jnp.float32)
            xr = _elu(agg + b)
            if i + 1 < n_layers:
                if reassoc[i + 1]:
                    dst[pl.ds(off, sz), 0:dout] = xr.astype(jnp.bfloat16)
                else:
                    s_next = jnp.dot(xr.astype(jnp.bfloat16), w_bf[i + 1],
                                     preferred_element_type=jnp.float32)
                    dst[pl.ds(off, sz), 0:dims[i + 1][1]] = (
                        s_next.astype(jnp.bfloat16))
            else:
                m = jnp.max(xr, axis=0, keepdims=True)
                acc = m if acc is None else jnp.maximum(acc, m)
    out_ref[...] = acc


def kernel(positions, adj, Ws, bs):
    bs2 = [b.reshape(1, -1) for b in bs]
    max_w = max(max(d) for d in (w.shape for w in Ws))
    n_in = 2 + len(Ws) + len(bs)
    in_specs = [pl.BlockSpec(memory_space=pltpu.MemorySpace.HBM) if i == 1
                else pl.BlockSpec(memory_space=pltpu.MemorySpace.VMEM)
                for i in range(n_in)]
    out = pl.pallas_call(
        _encoder_kernel,
        out_shape=jax.ShapeDtypeStruct((1, Ws[-1].shape[1]), jnp.float32),
        in_specs=in_specs,
        out_specs=pl.BlockSpec(memory_space=pltpu.MemorySpace.VMEM),
        scratch_shapes=[
            pltpu.VMEM((_N, _N), jnp.bfloat16),
            pltpu.VMEM((_N, _pad128(max_w)), jnp.bfloat16),
            pltpu.VMEM((_N, _pad128(max_w)), jnp.bfloat16),
            pltpu.VMEM((_CHUNK, _N), jnp.float32),
            pltpu.VMEM((_CHUNK, _N), jnp.float32),
            pltpu.VMEM((_N - (_N // _CHUNK) * _CHUNK, _N), jnp.float32),
            pltpu.SemaphoreType.DMA((3,)),
        ],
        compiler_params=pltpu.CompilerParams(
            vmem_limit_bytes=128 * 1024 * 1024,
        ),
    )(positions, adj, *Ws, *bs2)
    return out.reshape(-1)
